# baseline (speedup 1.0000x reference)
import numpy as np
import jax
from jax import lax
import jax.numpy as jnp
from jax.experimental import pallas as pl
from jax.experimental.pallas import tpu as pltpu
from jax.experimental.pallas import tpu_sc as plsc

EMBED_DIM = 64
CHUNK = 128
NBUF = 5
NUM_CORES = 2
NUM_SUBCORES = 16
NUM_WORKERS = NUM_CORES * NUM_SUBCORES
VEC = 16


def _permute_indices(idx_t, perm_s, perm_b):
    seq, batch = idx_t.shape
    num_indices = batch * seq
    per_worker = num_indices // NUM_WORKERS
    batches_per_worker = batch // NUM_WORKERS

    mesh = plsc.VectorSubcoreMesh(core_axis_name="c", subcore_axis_name="s")

    @pl.kernel(
        out_type=jax.ShapeDtypeStruct((num_indices,), jnp.int32),
        mesh=mesh,
        scratch_types=[
            pltpu.VMEM((seq, batches_per_worker), jnp.int32),
            pltpu.VMEM((per_worker,), jnp.int32),
            pltpu.VMEM((per_worker,), jnp.int32),
            pltpu.VMEM((per_worker,), jnp.int32),
            pltpu.SemaphoreType.DMA,
        ],
        compiler_params=pltpu.CompilerParams(
            use_tc_tiling_on_sc=False, needs_layout_passes=False
        ),
    )
    def permute_kernel(idxt_hbm, ps_hbm, pb_hbm, out_hbm,
                       blk_v, ps_v, pb_v, lin_v, sem):
        wid = lax.axis_index("s") * NUM_CORES + lax.axis_index("c")
        b0 = wid * batches_per_worker
        base = wid * per_worker

        pltpu.async_copy(
            idxt_hbm.at[:, pl.ds(b0, batches_per_worker)], blk_v, sem
        ).wait()
        pltpu.async_copy(ps_hbm, ps_v, sem).wait()
        pltpu.async_copy(pb_hbm, pb_v, sem).wait()

        @pl.loop(0, per_worker, step=VEC)
        def _(k0):
            sv = ps_v[pl.ds(k0, VEC)]
            bv = pb_v[pl.ds(k0, VEC)]
            lin_v[pl.ds(k0, VEC)] = plsc.load_gather(blk_v, [sv, bv])

        pltpu.async_copy(lin_v, out_hbm.at[pl.ds(base, per_worker)], sem).wait()

    return permute_kernel(idx_t, perm_s, perm_b)


def _transpose_table(table):
    t = table.T
    feat, vocab = t.shape
    bv = 512

    def body(x_ref, o_ref):
        o_ref[:, 0:feat] = x_ref[...].T

    return pl.pallas_call(
        body,
        grid=(pl.cdiv(vocab, bv),),
        in_specs=[pl.BlockSpec((feat, bv), lambda i: (0, i))],
        out_specs=pl.BlockSpec((bv, 128), lambda i: (i, 0)),
        out_shape=jax.ShapeDtypeStruct((vocab, 128), table.dtype),
    )(t)


def _gather_rows(table, indices, batch, seq):
    table = _transpose_table(table)
    num_indices = indices.shape[0]
    per_worker = num_indices // NUM_WORKERS
    bchunk = 4
    chunk = bchunk * seq
    num_chunks = per_worker // chunk
    nbuf = 4
    num_groups = num_chunks // nbuf
    batches_per_worker = batch // NUM_WORKERS

    mesh = plsc.VectorSubcoreMesh(core_axis_name="c", subcore_axis_name="s")

    @pl.kernel(
        out_type=jax.ShapeDtypeStruct((batch, seq, 128), table.dtype),
        mesh=mesh,
        scratch_types=[
            pltpu.VMEM((per_worker,), jnp.int32),
            pltpu.VMEM((nbuf, chunk, 128), jnp.float32),
            pltpu.SemaphoreType.DMA((nbuf,)),
            pltpu.SemaphoreType.DMA((nbuf,)),
            pltpu.SemaphoreType.DMA,
        ],
        compiler_params=pltpu.CompilerParams(use_tc_tiling_on_sc=True),
    )
    def gather_kernel(table_hbm, idx_hbm, out_hbm, idx_v, rows_v, gsem, osem, isem):
        wid = lax.axis_index("s") * NUM_CORES + lax.axis_index("c")
        base = wid * per_worker
        bbase = wid * batches_per_worker
        pltpu.async_copy(idx_hbm.at[pl.ds(base, per_worker)], idx_v, isem).wait()

        def gathers(c, b):
            return [
                pltpu.make_async_copy(
                    table_hbm.at[idx_v.at[pl.ds(c * chunk, 128)]],
                    rows_v.at[b, pl.ds(0, 128)],
                    gsem.at[b],
                ),
                pltpu.make_async_copy(
                    table_hbm.at[idx_v.at[pl.ds(c * chunk + 128, chunk - 128)]],
                    rows_v.at[b, pl.ds(128, chunk - 128)],
                    gsem.at[b],
                ),
            ]

        def puts(c, b):
            return [
                pltpu.make_async_copy(
                    rows_v.at[b, pl.ds(j * seq, seq)],
                    out_hbm.at[bbase + c * bchunk + j],
                    osem.at[b],
                )
                for j in range(bchunk)
            ]

        def start(ops):
            for op in ops:
                op.start()

        def wait(ops):
            for op in ops:
                op.wait()

        for b in range(nbuf):
            start(gathers(b, b))

        @pl.loop(0, num_groups - 1)
        def _(g):
            for b in range(nbuf):
                c = g * nbuf + b
                wait(gathers(c, b))
                start(puts(c, b))
            for b in range(nbuf):
                c = g * nbuf + b
                wait(puts(c, b))
                start(gathers(c + nbuf, b))

        for b in range(nbuf):
            c = (num_groups - 1) * nbuf + b
            wait(gathers(c, b))
            start(puts(c, b))
        for b in range(nbuf):
            c = (num_groups - 1) * nbuf + b
            wait(puts(c, b))

    return gather_kernel(table, indices)[:, :, :EMBED_DIM]


def kernel(input_seqs, table):
    batch, seq = input_seqs.shape
    num_indices = batch * seq
    per_worker = num_indices // NUM_WORKERS

    idx_t = input_seqs.T.astype(jnp.int32)

    k = np.arange(per_worker)
    perm_s = jnp.asarray((k % seq).astype(np.int32))
    perm_b = jnp.asarray((k // seq).astype(np.int32))

    indices = _permute_indices(idx_t, perm_s, perm_b)
    return _gather_rows(table, indices, batch, seq)


# speedup vs baseline: 1.6778x; 1.6778x over previous
import numpy as np
import jax
from jax import lax
import jax.numpy as jnp
from jax.experimental import pallas as pl
from jax.experimental.pallas import tpu as pltpu
from jax.experimental.pallas import tpu_sc as plsc

EMBED_DIM = 64
CHUNK = 128
NBUF = 5
NUM_CORES = 2
NUM_SUBCORES = 16
NUM_WORKERS = NUM_CORES * NUM_SUBCORES
VEC = 16


def _permute_indices(idx_t, perm_s, perm_b):
    seq, batch = idx_t.shape
    num_indices = batch * seq
    per_worker = num_indices // NUM_WORKERS
    batches_per_worker = batch // NUM_WORKERS

    mesh = plsc.VectorSubcoreMesh(core_axis_name="c", subcore_axis_name="s")

    @pl.kernel(
        out_type=jax.ShapeDtypeStruct((num_indices,), jnp.int32),
        mesh=mesh,
        scratch_types=[
            pltpu.VMEM((seq, batches_per_worker), jnp.int32),
            pltpu.VMEM((per_worker,), jnp.int32),
            pltpu.VMEM((per_worker,), jnp.int32),
            pltpu.VMEM((per_worker,), jnp.int32),
            pltpu.SemaphoreType.DMA,
        ],
        compiler_params=pltpu.CompilerParams(
            use_tc_tiling_on_sc=False, needs_layout_passes=False
        ),
    )
    def permute_kernel(idxt_hbm, ps_hbm, pb_hbm, out_hbm,
                       blk_v, ps_v, pb_v, lin_v, sem):
        wid = lax.axis_index("s") * NUM_CORES + lax.axis_index("c")
        b0 = wid * batches_per_worker
        base = wid * per_worker

        pltpu.async_copy(
            idxt_hbm.at[:, pl.ds(b0, batches_per_worker)], blk_v, sem
        ).wait()
        pltpu.async_copy(ps_hbm, ps_v, sem).wait()
        pltpu.async_copy(pb_hbm, pb_v, sem).wait()

        @pl.loop(0, per_worker, step=VEC)
        def _(k0):
            sv = ps_v[pl.ds(k0, VEC)]
            bv = pb_v[pl.ds(k0, VEC)]
            lin_v[pl.ds(k0, VEC)] = plsc.load_gather(blk_v, [sv, bv])

        pltpu.async_copy(lin_v, out_hbm.at[pl.ds(base, per_worker)], sem).wait()

    return permute_kernel(idx_t, perm_s, perm_b)


def _widen_table(table):
    vocab, feat = table.shape
    bv = 8192

    def body(x_ref, o_ref):
        o_ref[:, 0:feat] = x_ref[...]

    return pl.pallas_call(
        body,
        grid=(pl.cdiv(vocab, bv),),
        in_specs=[pl.BlockSpec((bv, feat), lambda i: (i, 0))],
        out_specs=pl.BlockSpec((bv, 128), lambda i: (i, 0)),
        out_shape=jax.ShapeDtypeStruct((vocab, 128), table.dtype),
    )(table)


def _gather_rows(table, indices, batch, seq):
    table = _widen_table(table)
    num_indices = indices.shape[0]
    per_worker = num_indices // NUM_WORKERS
    bchunk = 4
    chunk = bchunk * seq
    num_chunks = per_worker // chunk
    nbuf = 4
    num_groups = num_chunks // nbuf
    batches_per_worker = batch // NUM_WORKERS

    mesh = plsc.VectorSubcoreMesh(core_axis_name="c", subcore_axis_name="s")

    @pl.kernel(
        out_type=jax.ShapeDtypeStruct((batch, seq, 128), table.dtype),
        mesh=mesh,
        scratch_types=[
            pltpu.VMEM((per_worker,), jnp.int32),
            pltpu.VMEM((nbuf, chunk, 128), jnp.float32),
            pltpu.SemaphoreType.DMA((nbuf,)),
            pltpu.SemaphoreType.DMA((nbuf,)),
            pltpu.SemaphoreType.DMA,
        ],
        compiler_params=pltpu.CompilerParams(use_tc_tiling_on_sc=True),
    )
    def gather_kernel(table_hbm, idx_hbm, out_hbm, idx_v, rows_v, gsem, osem, isem):
        wid = lax.axis_index("s") * NUM_CORES + lax.axis_index("c")
        base = wid * per_worker
        bbase = wid * batches_per_worker
        pltpu.async_copy(idx_hbm.at[pl.ds(base, per_worker)], idx_v, isem).wait()

        def gathers(c, b):
            return [
                pltpu.make_async_copy(
                    table_hbm.at[idx_v.at[pl.ds(c * chunk, 128)]],
                    rows_v.at[b, pl.ds(0, 128)],
                    gsem.at[b],
                ),
                pltpu.make_async_copy(
                    table_hbm.at[idx_v.at[pl.ds(c * chunk + 128, chunk - 128)]],
                    rows_v.at[b, pl.ds(128, chunk - 128)],
                    gsem.at[b],
                ),
            ]

        def puts(c, b):
            return [
                pltpu.make_async_copy(
                    rows_v.at[b, pl.ds(j * seq, seq)],
                    out_hbm.at[bbase + c * bchunk + j],
                    osem.at[b],
                )
                for j in range(bchunk)
            ]

        def start(ops):
            for op in ops:
                op.start()

        def wait(ops):
            for op in ops:
                op.wait()

        for b in range(nbuf):
            start(gathers(b, b))

        @pl.loop(0, num_groups - 1)
        def _(g):
            for b in range(nbuf):
                c = g * nbuf + b
                wait(gathers(c, b))
                start(puts(c, b))
            for b in range(nbuf):
                c = g * nbuf + b
                wait(puts(c, b))
                start(gathers(c + nbuf, b))

        for b in range(nbuf):
            c = (num_groups - 1) * nbuf + b
            wait(gathers(c, b))
            start(puts(c, b))
        for b in range(nbuf):
            c = (num_groups - 1) * nbuf + b
            wait(puts(c, b))

    return gather_kernel(table, indices)[:, :, :EMBED_DIM]


def kernel(input_seqs, table):
    batch, seq = input_seqs.shape
    num_indices = batch * seq
    per_worker = num_indices // NUM_WORKERS

    idx_t = input_seqs.T.astype(jnp.int32)

    k = np.arange(per_worker)
    perm_s = jnp.asarray((k % seq).astype(np.int32))
    perm_b = jnp.asarray((k // seq).astype(np.int32))

    indices = _permute_indices(idx_t, perm_s, perm_b)
    return _gather_rows(table, indices, batch, seq)


# speedup vs baseline: 1.9956x; 1.1894x over previous
import numpy as np
import jax
from jax import lax
import jax.numpy as jnp
from jax.experimental import pallas as pl
from jax.experimental.pallas import tpu as pltpu
from jax.experimental.pallas import tpu_sc as plsc

EMBED_DIM = 64
CHUNK = 128
NBUF = 5
NUM_CORES = 2
NUM_SUBCORES = 16
NUM_WORKERS = NUM_CORES * NUM_SUBCORES
VEC = 16


def _permute_indices(idx_t, perm_s, perm_b):
    seq, batch = idx_t.shape
    num_indices = batch * seq
    per_worker = num_indices // NUM_WORKERS
    batches_per_worker = batch // NUM_WORKERS

    mesh = plsc.VectorSubcoreMesh(core_axis_name="c", subcore_axis_name="s")

    @pl.kernel(
        out_type=jax.ShapeDtypeStruct((num_indices,), jnp.int32),
        mesh=mesh,
        scratch_types=[
            pltpu.VMEM((seq, batches_per_worker), jnp.int32),
            pltpu.VMEM((per_worker,), jnp.int32),
            pltpu.VMEM((per_worker,), jnp.int32),
            pltpu.VMEM((per_worker,), jnp.int32),
            pltpu.SemaphoreType.DMA,
        ],
        compiler_params=pltpu.CompilerParams(
            use_tc_tiling_on_sc=False, needs_layout_passes=False
        ),
    )
    def permute_kernel(idxt_hbm, ps_hbm, pb_hbm, out_hbm,
                       blk_v, ps_v, pb_v, lin_v, sem):
        wid = lax.axis_index("s") * NUM_CORES + lax.axis_index("c")
        b0 = wid * batches_per_worker
        base = wid * per_worker

        pltpu.async_copy(
            idxt_hbm.at[:, pl.ds(b0, batches_per_worker)], blk_v, sem
        ).wait()
        pltpu.async_copy(ps_hbm, ps_v, sem).wait()
        pltpu.async_copy(pb_hbm, pb_v, sem).wait()

        @pl.loop(0, per_worker, step=VEC)
        def _(k0):
            sv = ps_v[pl.ds(k0, VEC)]
            bv = pb_v[pl.ds(k0, VEC)]
            lin_v[pl.ds(k0, VEC)] = plsc.load_gather(blk_v, [sv, bv])

        pltpu.async_copy(lin_v, out_hbm.at[pl.ds(base, per_worker)], sem).wait()

    return permute_kernel(idx_t, perm_s, perm_b)


def _gather_rows(table, indices, batch, seq):
    table = jnp.pad(table, ((0, 0), (0, 128 - EMBED_DIM)))
    num_indices = indices.shape[0]
    per_worker = num_indices // NUM_WORKERS
    bchunk = 4
    chunk = bchunk * seq
    num_chunks = per_worker // chunk
    nbuf = 4
    num_groups = num_chunks // nbuf
    batches_per_worker = batch // NUM_WORKERS

    mesh = plsc.VectorSubcoreMesh(core_axis_name="c", subcore_axis_name="s")

    @pl.kernel(
        out_type=jax.ShapeDtypeStruct((batch, seq, 128), table.dtype),
        mesh=mesh,
        scratch_types=[
            pltpu.VMEM((per_worker,), jnp.int32),
            pltpu.VMEM((nbuf, chunk, 128), jnp.float32),
            pltpu.SemaphoreType.DMA((nbuf,)),
            pltpu.SemaphoreType.DMA((nbuf,)),
            pltpu.SemaphoreType.DMA,
        ],
        compiler_params=pltpu.CompilerParams(use_tc_tiling_on_sc=True),
    )
    def gather_kernel(table_hbm, idx_hbm, out_hbm, idx_v, rows_v, gsem, osem, isem):
        wid = lax.axis_index("s") * NUM_CORES + lax.axis_index("c")
        base = wid * per_worker
        bbase = wid * batches_per_worker
        pltpu.async_copy(idx_hbm.at[pl.ds(base, per_worker)], idx_v, isem).wait()

        def gathers(c, b):
            return [
                pltpu.make_async_copy(
                    table_hbm.at[idx_v.at[pl.ds(c * chunk, 128)]],
                    rows_v.at[b, pl.ds(0, 128)],
                    gsem.at[b],
                ),
                pltpu.make_async_copy(
                    table_hbm.at[idx_v.at[pl.ds(c * chunk + 128, chunk - 128)]],
                    rows_v.at[b, pl.ds(128, chunk - 128)],
                    gsem.at[b],
                ),
            ]

        def puts(c, b):
            return [
                pltpu.make_async_copy(
                    rows_v.at[b, pl.ds(j * seq, seq)],
                    out_hbm.at[bbase + c * bchunk + j],
                    osem.at[b],
                )
                for j in range(bchunk)
            ]

        def start(ops):
            for op in ops:
                op.start()

        def wait(ops):
            for op in ops:
                op.wait()

        for b in range(nbuf):
            start(gathers(b, b))

        @pl.loop(0, num_groups - 1)
        def _(g):
            for b in range(nbuf):
                c = g * nbuf + b
                wait(gathers(c, b))
                start(puts(c, b))
            for b in range(nbuf):
                c = g * nbuf + b
                wait(puts(c, b))
                start(gathers(c + nbuf, b))

        for b in range(nbuf):
            c = (num_groups - 1) * nbuf + b
            wait(gathers(c, b))
            start(puts(c, b))
        for b in range(nbuf):
            c = (num_groups - 1) * nbuf + b
            wait(puts(c, b))

    return gather_kernel(table, indices)[:, :, :EMBED_DIM]


def kernel(input_seqs, table):
    batch, seq = input_seqs.shape
    num_indices = batch * seq
    per_worker = num_indices // NUM_WORKERS

    idx_t = input_seqs.T.astype(jnp.int32)

    k = np.arange(per_worker)
    perm_s = jnp.asarray((k % seq).astype(np.int32))
    perm_b = jnp.asarray((k // seq).astype(np.int32))

    indices = _permute_indices(idx_t, perm_s, perm_b)
    return _gather_rows(table, indices, batch, seq)


# speedup vs baseline: 2.0014x; 1.0029x over previous
import numpy as np
import jax
from jax import lax
import jax.numpy as jnp
from jax.experimental import pallas as pl
from jax.experimental.pallas import tpu as pltpu
from jax.experimental.pallas import tpu_sc as plsc

EMBED_DIM = 64
NUM_CORES = 2
NUM_SUBCORES = 16
NUM_WORKERS = NUM_CORES * NUM_SUBCORES
VEC = 16


def _permute_indices(idx_t, perm_s, perm_b):
    seq, batch = idx_t.shape
    num_indices = batch * seq
    per_worker = num_indices // NUM_WORKERS
    batches_per_worker = batch // NUM_WORKERS

    mesh = plsc.VectorSubcoreMesh(core_axis_name="c", subcore_axis_name="s")

    @pl.kernel(
        out_type=jax.ShapeDtypeStruct((num_indices,), jnp.int32),
        mesh=mesh,
        scratch_types=[
            pltpu.VMEM((seq, batches_per_worker), jnp.int32),
            pltpu.VMEM((per_worker,), jnp.int32),
            pltpu.VMEM((per_worker,), jnp.int32),
            pltpu.VMEM((per_worker,), jnp.int32),
            pltpu.SemaphoreType.DMA,
        ],
        compiler_params=pltpu.CompilerParams(
            use_tc_tiling_on_sc=False, needs_layout_passes=False
        ),
    )
    def permute_kernel(idxt_hbm, ps_hbm, pb_hbm, out_hbm,
                       blk_v, ps_v, pb_v, lin_v, sem):
        wid = lax.axis_index("s") * NUM_CORES + lax.axis_index("c")
        b0 = wid * batches_per_worker
        base = wid * per_worker

        pltpu.async_copy(
            idxt_hbm.at[:, pl.ds(b0, batches_per_worker)], blk_v, sem
        ).wait()
        pltpu.async_copy(ps_hbm, ps_v, sem).wait()
        pltpu.async_copy(pb_hbm, pb_v, sem).wait()

        @pl.loop(0, per_worker, step=VEC)
        def _(k0):
            sv = ps_v[pl.ds(k0, VEC)]
            bv = pb_v[pl.ds(k0, VEC)]
            lin_v[pl.ds(k0, VEC)] = plsc.load_gather(blk_v, [sv, bv])

        pltpu.async_copy(lin_v, out_hbm.at[pl.ds(base, per_worker)], sem).wait()

    return permute_kernel(idx_t, perm_s, perm_b)


def _gather_rows(table, indices, batch, seq):
    table = jnp.pad(table, ((0, 0), (0, 128 - EMBED_DIM)))
    num_indices = indices.shape[0]
    per_worker = num_indices // NUM_WORKERS
    bchunk = 4
    chunk = bchunk * seq
    num_chunks = per_worker // chunk
    nbuf = 4
    num_groups = num_chunks // nbuf
    batches_per_worker = batch // NUM_WORKERS

    mesh = plsc.VectorSubcoreMesh(core_axis_name="c", subcore_axis_name="s")

    @pl.kernel(
        out_type=jax.ShapeDtypeStruct((batch, seq, 128), table.dtype),
        mesh=mesh,
        scratch_types=[
            pltpu.VMEM((per_worker,), jnp.int32),
            pltpu.VMEM((nbuf, chunk, 128), jnp.float32),
            pltpu.SemaphoreType.DMA((nbuf,)),
            pltpu.SemaphoreType.DMA((nbuf,)),
            pltpu.SemaphoreType.DMA,
        ],
        compiler_params=pltpu.CompilerParams(use_tc_tiling_on_sc=True),
    )
    def gather_kernel(table_hbm, idx_hbm, out_hbm, idx_v, rows_v, gsem, osem, isem):
        wid = lax.axis_index("s") * NUM_CORES + lax.axis_index("c")
        base = wid * per_worker
        bbase = wid * batches_per_worker
        pltpu.async_copy(idx_hbm.at[pl.ds(base, per_worker)], idx_v, isem).wait()

        def gathers(c, b):
            return [
                pltpu.make_async_copy(
                    table_hbm.at[idx_v.at[pl.ds(c * chunk, 128)]],
                    rows_v.at[b, pl.ds(0, 128)],
                    gsem.at[b],
                ),
                pltpu.make_async_copy(
                    table_hbm.at[idx_v.at[pl.ds(c * chunk + 128, chunk - 128)]],
                    rows_v.at[b, pl.ds(128, chunk - 128)],
                    gsem.at[b],
                ),
            ]

        def puts(c, b):
            return [
                pltpu.make_async_copy(
                    rows_v.at[b, pl.ds(j * seq, seq)],
                    out_hbm.at[bbase + c * bchunk + j],
                    osem.at[b],
                )
                for j in range(bchunk)
            ]

        def start(ops):
            for op in ops:
                op.start()

        def wait(ops):
            for op in ops:
                op.wait()

        for b in range(nbuf):
            start(gathers(b, b))

        @pl.loop(0, num_groups - 1)
        def _(g):
            for b in range(nbuf):
                c = g * nbuf + b
                wait(gathers(c, b))
                start(puts(c, b))
            for b in range(nbuf):
                c = g * nbuf + b
                wait(puts(c, b))
                start(gathers(c + nbuf, b))

        for b in range(nbuf):
            c = (num_groups - 1) * nbuf + b
            wait(gathers(c, b))
            start(puts(c, b))
        for b in range(nbuf):
            c = (num_groups - 1) * nbuf + b
            wait(puts(c, b))

    return gather_kernel(table, indices)[:, :, :EMBED_DIM]


def kernel(input_seqs, table):
    batch, seq = input_seqs.shape
    num_indices = batch * seq
    per_worker = num_indices // NUM_WORKERS

    idx_t = input_seqs.T.astype(jnp.int32)

    k = np.arange(per_worker)
    perm_s = jnp.asarray((k % seq).astype(np.int32))
    perm_b = jnp.asarray((k // seq).astype(np.int32))

    indices = _permute_indices(idx_t, perm_s, perm_b)
    return _gather_rows(table, indices, batch, seq)
